# baseline (speedup 1.0000x reference)
import functools

import jax
import jax.numpy as jnp
from jax import lax
from jax.experimental import pallas as pl
from jax.experimental.pallas import tpu as pltpu
from jax.experimental.pallas import tpu_sc as plsc

N = 100000
E = 1600000
RAW, EMB, HID, HEADS, LSTMH, NA, NO = 11, 32, 64, 8, 64, 7, 2
DH = HID // HEADS

ROWS = 4000
GRID = N // ROWS

NC, NS = 2, 16
NPART = 16
PSIZE = 6256
NPAD = NPART * PSIZE
APAD = 6272
PCHUNK = 392
PLAST = 376
DUMP = PSIZE + 4
EPW = E // (NC * NS)
EB = 2000
NBLK = EPW // EB
VPB = EB // 16
HPAD = 100096
HCH = HPAD // NS
DEB = 2000
DVPB = DEB // 16

_MESH = plsc.VectorSubcoreMesh(core_axis_name="c", subcore_axis_name="s",
                               num_cores=NC, num_subcores=NS)
_SC_PARAMS = pltpu.CompilerParams(needs_layout_passes=False,
                                  use_tc_tiling_on_sc=False)


@functools.partial(
    pl.kernel,
    out_type=[jax.ShapeDtypeStruct((NC * HPAD,), jnp.float32),
              jax.ShapeDtypeStruct((NC * NS * HPAD,), jnp.float32)],
    mesh=_MESH,
    compiler_params=_SC_PARAMS,
    scratch_types=dict(
        hist=pltpu.VMEM((HPAD,), jnp.float32),
        dstv=pltpu.VMEM((DEB,), jnp.int32),
        accv=pltpu.VMEM((HCH,), jnp.float32),
        tmpv=pltpu.VMEM((HCH,), jnp.float32),
    ),
)
def _deg_kernel(dst_hbm, degp_hbm, part_hbm, hist, dstv, accv, tmpv):
    cid = lax.axis_index("c")
    sid = lax.axis_index("s")
    wid = sid * NC + cid
    zeros16 = jnp.zeros((16,), jnp.float32)
    ones16 = jnp.ones((16,), jnp.float32)

    def zero_body(i, _):
        hist[pl.ds(i * 16, 16)] = zeros16
        return 0

    lax.fori_loop(0, HPAD // 16, zero_body, 0)

    def blk_body(b, _):
        pltpu.sync_copy(dst_hbm.at[pl.ds(wid * EPW + b * DEB, DEB)], dstv)

        def vec_body(t, _):
            d16 = dstv[pl.ds(t * 16, 16)]
            plsc.addupdate_scatter(hist, [d16], ones16)
            return 0

        return lax.fori_loop(0, DVPB, vec_body, 0)

    lax.fori_loop(0, EPW // DEB, blk_body, 0)

    pltpu.sync_copy(hist, part_hbm.at[pl.ds((cid * NS + sid) * HPAD, HPAD)])
    plsc.subcore_barrier()

    col = sid * HCH
    pltpu.sync_copy(part_hbm.at[pl.ds(cid * NS * HPAD + col, HCH)], accv)
    for j in range(1, NS):
        pltpu.sync_copy(part_hbm.at[pl.ds((cid * NS + j) * HPAD + col, HCH)],
                        tmpv)

        def add_body(v, _):
            accv[pl.ds(v * 16, 16)] = accv[pl.ds(v * 16, 16)] + tmpv[pl.ds(v * 16, 16)]
            return 0

        lax.fori_loop(0, HCH // 16, add_body, 0)
    pltpu.sync_copy(accv, degp_hbm.at[pl.ds(cid * HPAD + col, HCH)])


@functools.partial(
    pl.kernel,
    out_type=jax.ShapeDtypeStruct((NC * NPAD, HID), jnp.float32),
    mesh=_MESH,
    compiler_params=_SC_PARAMS,
    scratch_types=dict(
        srcv=pltpu.VMEM((EB,), jnp.int32),
        dstv=pltpu.VMEM((EB,), jnp.int32),
        stage_s=pltpu.VMEM((160,), jnp.int32),
        stage_o=pltpu.VMEM((160,), jnp.int32),
        gidx=pltpu.VMEM((1, 128), jnp.int32),
        goff=pltpu.VMEM((1, 128), jnp.int32),
        rows=pltpu.VMEM((128, HID), jnp.float32),
        acc=pltpu.VMEM_SHARED((APAD, HID), jnp.float32),
    ),
)
def _gcn_agg_kernel(xs_hbm, src_hbm, dst_hbm, zeros_hbm, out_hbm,
                    srcv, dstv, stage_s, stage_o, gidx, goff, rows, acc):
    cid = lax.axis_index("c")
    sid = lax.axis_index("s")
    lanes = lax.iota(jnp.int32, 16)
    cbase = cid * NPAD
    ebase0 = (cid * NS + sid) * EPW

    for part in range(NPART):
        base = part * PSIZE

        pltpu.sync_copy(zeros_hbm.at[pl.ds(sid * PCHUNK, PCHUNK)],
                        acc.at[pl.ds(sid * PCHUNK, PCHUNK)])
        plsc.subcore_barrier()

        def fire(n_valid):
            for i in range(8):
                sv = stage_s[pl.ds(i * 16, 16)]
                ov = stage_o[pl.ds(i * 16, 16)]
                valid = (lanes + (i * 16)) < n_valid
                gidx[0, pl.ds(i * 16, 16)] = jnp.where(valid, sv, 0)
                goff[0, pl.ds(i * 16, 16)] = jnp.where(valid, ov, DUMP)
            pltpu.sync_copy(xs_hbm.at[gidx.at[0]], rows)
            pltpu.sync_copy(rows, acc.at[goff.at[0]], add=True)

        def blk_body(b, ptr):
            ebase = ebase0 + b * EB
            pltpu.sync_copy(src_hbm.at[pl.ds(ebase, EB)], srcv)
            pltpu.sync_copy(dst_hbm.at[pl.ds(ebase, EB)], dstv)

            def vec_body(t, ptr):
                s16 = srcv[pl.ds(t * 16, 16)]
                d16 = dstv[pl.ds(t * 16, 16)]
                m = (d16 >= base) & (d16 < base + PSIZE)
                plsc.store_compressed(stage_s.at[pl.ds(ptr, 16)], s16, mask=m)
                plsc.store_compressed(stage_o.at[pl.ds(ptr, 16)], d16 - base,
                                      mask=m)
                ptr = ptr + jnp.sum(m.astype(jnp.int32))

                def do_fire(p):
                    fire(jnp.int32(128))
                    rs = stage_s[pl.ds(128, 16)]
                    ro = stage_o[pl.ds(128, 16)]
                    stage_s[pl.ds(0, 16)] = rs
                    stage_o[pl.ds(0, 16)] = ro
                    return p - 128

                return lax.cond(ptr >= 128, do_fire, lambda p: p, ptr)

            return lax.fori_loop(0, VPB, vec_body, ptr)

        ptr = lax.fori_loop(0, NBLK, blk_body, jnp.int32(0))

        def tail(p):
            fire(p)
            return jnp.int32(0)

        ptr = lax.cond(ptr > 0, tail, lambda p: jnp.int32(0), ptr)
        plsc.subcore_barrier()

        @pl.when(sid < NS - 1)
        def _():
            pltpu.sync_copy(acc.at[pl.ds(sid * PCHUNK, PCHUNK)],
                            out_hbm.at[pl.ds(cbase + base + sid * PCHUNK,
                                             PCHUNK)])

        @pl.when(sid == NS - 1)
        def _():
            pltpu.sync_copy(acc.at[pl.ds((NS - 1) * PCHUNK, PLAST)],
                            out_hbm.at[pl.ds(cbase + base + (NS - 1) * PCHUNK,
                                             PLAST)])

        plsc.subcore_barrier()


_CW = 128


_GDN = lax.GatherDimensionNumbers(offset_dims=(), collapsed_slice_dims=(0,),
                                  start_index_map=(0,))


def _perm(x, idx):
    return lax.gather(x, idx.reshape(16, 1), _GDN, slice_sizes=(1,),
                      mode=lax.GatherScatterMode.PROMISE_IN_BOUNDS)


@functools.partial(
    pl.kernel,
    out_type=jax.ShapeDtypeStruct((NC * NPAD, _CW), jnp.float32),
    mesh=_MESH,
    compiler_params=_SC_PARAMS,
    scratch_types=dict(
        srcv=pltpu.VMEM((EB,), jnp.int32),
        dstv=pltpu.VMEM((EB,), jnp.int32),
        stage_s=pltpu.VMEM((160,), jnp.int32),
        stage_d=pltpu.VMEM((160,), jnp.int32),
        gsrc=pltpu.VMEM((1, 128), jnp.int32),
        gdst=pltpu.VMEM((1, 128), jnp.int32),
        goff=pltpu.VMEM((1, 128), jnp.int32),
        qrows=pltpu.VMEM((128, HID), jnp.float32),
        krows=pltpu.VMEM((128, HID), jnp.float32),
        vrows=pltpu.VMEM((128, HID), jnp.float32),
        cat=pltpu.VMEM((128, _CW), jnp.float32),
        acc=pltpu.VMEM_SHARED((APAD, _CW), jnp.float32),
    ),
)
def _attn_kernel(q_hbm, k_hbm, v_hbm, src_hbm, dst_hbm, zeros_hbm, out_hbm,
                 srcv, dstv, stage_s, stage_d, gsrc, gdst, goff,
                 qrows, krows, vrows, cat, acc):
    cid = lax.axis_index("c")
    sid = lax.axis_index("s")
    lanes = lax.iota(jnp.int32, 16)
    scale = jnp.float32(1.0 / (DH ** 0.5))
    perm_sw4 = lanes ^ 4
    perm_sw2 = lanes ^ 2
    perm_sw1 = lanes ^ 1
    cbase = cid * NPAD
    ebase0 = (cid * NS + sid) * EPW

    def edge_body(e, _):
        for j in range(4):
            qj = qrows[e, pl.ds(j * 16, 16)]
            kj = krows[e, pl.ds(j * 16, 16)]
            t = qj * kj
            t = t + _perm(t, perm_sw4)
            t = t + _perm(t, perm_sw2)
            t = t + _perm(t, perm_sw1)
            ej = jnp.exp(jnp.minimum(t * scale, 70.0))
            cat[e, pl.ds(j * 16, 16)] = ej * vrows[e, pl.ds(j * 16, 16)]
            cat[e, pl.ds(64 + j * 16, 16)] = ej
        return 0

    for part in range(NPART):
        base = part * PSIZE

        pltpu.sync_copy(zeros_hbm.at[pl.ds(sid * PCHUNK, PCHUNK)],
                        acc.at[pl.ds(sid * PCHUNK, PCHUNK)])
        plsc.subcore_barrier()

        def fire(n_valid):
            for i in range(8):
                sv = stage_s[pl.ds(i * 16, 16)]
                dv = stage_d[pl.ds(i * 16, 16)]
                valid = (lanes + (i * 16)) < n_valid
                gsrc[0, pl.ds(i * 16, 16)] = jnp.where(valid, sv, 0)
                gdst[0, pl.ds(i * 16, 16)] = jnp.where(valid, dv, 0)
                goff[0, pl.ds(i * 16, 16)] = jnp.where(valid, dv - base, DUMP)
            pltpu.sync_copy(q_hbm.at[gdst.at[0]], qrows)
            pltpu.sync_copy(k_hbm.at[gsrc.at[0]], krows)
            pltpu.sync_copy(v_hbm.at[gsrc.at[0]], vrows)
            lax.fori_loop(0, 128, edge_body, 0)
            pltpu.sync_copy(cat, acc.at[goff.at[0]], add=True)

        def blk_body(b, ptr):
            ebase = ebase0 + b * EB
            pltpu.sync_copy(src_hbm.at[pl.ds(ebase, EB)], srcv)
            pltpu.sync_copy(dst_hbm.at[pl.ds(ebase, EB)], dstv)

            def vec_body(t, ptr):
                s16 = srcv[pl.ds(t * 16, 16)]
                d16 = dstv[pl.ds(t * 16, 16)]
                m = (d16 >= base) & (d16 < base + PSIZE)
                plsc.store_compressed(stage_s.at[pl.ds(ptr, 16)], s16, mask=m)
                plsc.store_compressed(stage_d.at[pl.ds(ptr, 16)], d16, mask=m)
                ptr = ptr + jnp.sum(m.astype(jnp.int32))

                def do_fire(p):
                    fire(jnp.int32(128))
                    rs = stage_s[pl.ds(128, 16)]
                    rd = stage_d[pl.ds(128, 16)]
                    stage_s[pl.ds(0, 16)] = rs
                    stage_d[pl.ds(0, 16)] = rd
                    return p - 128

                return lax.cond(ptr >= 128, do_fire, lambda p: p, ptr)

            return lax.fori_loop(0, VPB, vec_body, ptr)

        ptr = lax.fori_loop(0, NBLK, blk_body, jnp.int32(0))

        def tail(p):
            fire(p)
            return jnp.int32(0)

        ptr = lax.cond(ptr > 0, tail, lambda p: jnp.int32(0), ptr)
        plsc.subcore_barrier()

        @pl.when(sid < NS - 1)
        def _():
            pltpu.sync_copy(acc.at[pl.ds(sid * PCHUNK, PCHUNK)],
                            out_hbm.at[pl.ds(cbase + base + sid * PCHUNK,
                                             PCHUNK)])

        @pl.when(sid == NS - 1)
        def _():
            pltpu.sync_copy(acc.at[pl.ds((NS - 1) * PCHUNK, PLAST)],
                            out_hbm.at[pl.ds(cbase + base + (NS - 1) * PCHUNK,
                                             PLAST)])

        plsc.subcore_barrier()


def _tc_a_body(nf_ref, d0_ref, d1_ref, w1_ref, b1_ref, w2_ref,
               h1_ref, xs_ref, dinv_ref, dinv2_ref):
    x = jnp.maximum(nf_ref[...] @ w1_ref[...] + b1_ref[...], 0.0)
    h1 = x @ w2_ref[...]
    deg = 1.0 + d0_ref[...] + d1_ref[...]
    dinv = lax.rsqrt(deg)
    h1_ref[...] = h1
    xs_ref[...] = dinv * h1
    dinv_ref[...] = dinv
    dinv2_ref[...] = dinv * dinv


def _tc_a(nf, d0, d1, w1, b1, w2):
    col = pl.BlockSpec((ROWS, 1), lambda i: (i, 0))
    row = pl.BlockSpec((ROWS, HID), lambda i: (i, 0))
    return pl.pallas_call(
        _tc_a_body,
        grid=(GRID,),
        in_specs=[
            pl.BlockSpec((ROWS, RAW), lambda i: (i, 0)), col, col,
            pl.BlockSpec((RAW, EMB), lambda i: (0, 0)),
            pl.BlockSpec((1, EMB), lambda i: (0, 0)),
            pl.BlockSpec((EMB, HID), lambda i: (0, 0)),
        ],
        out_specs=[row, row, col, col],
        out_shape=[
            jax.ShapeDtypeStruct((N, HID), jnp.float32),
            jax.ShapeDtypeStruct((N, HID), jnp.float32),
            jax.ShapeDtypeStruct((N, 1), jnp.float32),
            jax.ShapeDtypeStruct((N, 1), jnp.float32),
        ],
    )(nf, d0, d1, w1, b1.reshape(1, EMB), w2)


def _tc_b_body(sa_ref, sb_ref, h_ref, dinv_ref, d2_ref, b_ref, w_ref,
               h2_ref, xs_ref):
    s = sa_ref[...] + sb_ref[...]
    t = jnp.maximum(dinv_ref[...] * s + d2_ref[...] * h_ref[...]
                    + b_ref[...], 0.0)
    h2 = t @ w_ref[...]
    h2_ref[...] = h2
    xs_ref[...] = dinv_ref[...] * h2


def _tc_b(Sa, Sb, h, dinv, dinv2, b, w2):
    col = pl.BlockSpec((ROWS, 1), lambda i: (i, 0))
    row = pl.BlockSpec((ROWS, HID), lambda i: (i, 0))
    return pl.pallas_call(
        _tc_b_body,
        grid=(GRID,),
        in_specs=[row, row, row, col, col,
                  pl.BlockSpec((1, HID), lambda i: (0, 0)),
                  pl.BlockSpec((HID, HID), lambda i: (0, 0))],
        out_specs=[row, row],
        out_shape=[jax.ShapeDtypeStruct((N, HID), jnp.float32)] * 2,
    )(Sa, Sb, h, dinv, dinv2, b.reshape(1, HID), w2)


def _tc_c_body(sa_ref, sb_ref, h_ref, dinv_ref, d2_ref, gb_ref,
               wq_ref, wk_ref, wv_ref, ws_ref, bq_ref, bk_ref, bv_ref,
               bs_ref, q_ref, k_ref, v_ref, sk_ref):
    s = sa_ref[...] + sb_ref[...]
    x = jnp.maximum(dinv_ref[...] * s + d2_ref[...] * h_ref[...]
                    + gb_ref[...], 0.0)
    q_ref[...] = x @ wq_ref[...] + bq_ref[...]
    k_ref[...] = x @ wk_ref[...] + bk_ref[...]
    v_ref[...] = x @ wv_ref[...] + bv_ref[...]
    sk_ref[...] = x @ ws_ref[...] + bs_ref[...]


def _tc_c(Sa, Sb, h, dinv, dinv2, gb, p):
    col = pl.BlockSpec((ROWS, 1), lambda i: (i, 0))
    row = pl.BlockSpec((ROWS, HID), lambda i: (i, 0))
    wspec = pl.BlockSpec((HID, HID), lambda i: (0, 0))
    bspec = pl.BlockSpec((1, HID), lambda i: (0, 0))
    return pl.pallas_call(
        _tc_c_body,
        grid=(GRID,),
        in_specs=[row, row, row, col, col, bspec,
                  wspec, wspec, wspec, wspec, bspec, bspec, bspec, bspec],
        out_specs=[row, row, row, row],
        out_shape=[jax.ShapeDtypeStruct((N, HID), jnp.float32)] * 4,
    )(Sa, Sb, h, dinv, dinv2, gb.reshape(1, HID),
      p['Wq'], p['Wk'], p['Wv'], p['Wskip'],
      p['bq'].reshape(1, HID), p['bk'].reshape(1, HID),
      p['bv'].reshape(1, HID), p['bskip'].reshape(1, HID))


def _tc_d_body(ca_ref, cb_ref, sk_ref, out_ref):
    @pl.when(pl.program_id(0) == 0)
    def _():
        out_ref[...] = jnp.zeros_like(out_ref)
    cat = ca_ref[...] + cb_ref[...]
    num = cat[:, :HID]
    den = cat[:, HID:]
    attn = num / (den + 1e-16)
    out_ref[...] += jnp.sum(attn + sk_ref[...], axis=0, keepdims=True)


def _tc_d(cata, catb, skip):
    return pl.pallas_call(
        _tc_d_body,
        grid=(GRID,),
        in_specs=[
            pl.BlockSpec((ROWS, _CW), lambda i: (i, 0)),
            pl.BlockSpec((ROWS, _CW), lambda i: (i, 0)),
            pl.BlockSpec((ROWS, HID), lambda i: (i, 0)),
        ],
        out_specs=pl.BlockSpec((1, HID), lambda i: (0, 0)),
        out_shape=jax.ShapeDtypeStruct((1, HID), jnp.float32),
    )(cata, catb, skip)


def _head(ge, h_prev, c_prev, w, p):
    ww = w.reshape(1, -1)
    g = ww @ p['film_gW'] + p['film_gb']
    b = ww @ p['film_bW'] + p['film_bb']
    ge = ge * (1.0 + g) + b
    gates = ge @ p['lstm_Wih'] + p['lstm_bih'] + h_prev @ p['lstm_Whh'] + p['lstm_bhh']
    i, f, gg, o = jnp.split(gates, 4, axis=-1)
    i = jax.nn.sigmoid(i)
    f = jax.nn.sigmoid(f)
    gg = jnp.tanh(gg)
    o = jax.nn.sigmoid(o)
    c_new = f * c_prev + i * gg
    h_new = o * jnp.tanh(c_new)
    mu = jnp.mean(h_new, axis=-1, keepdims=True)
    var = jnp.var(h_new, axis=-1, keepdims=True)
    h_new = (h_new - mu) / jnp.sqrt(var + 1e-5) * p['ln_g'] + p['ln_b']
    c_new = jnp.clip(c_new, -1e6, 1e6)
    val = jax.nn.relu(h_new @ p['val_W1'] + p['val_b1']) @ p['val_W2'] + p['val_b2']
    adv = (jax.nn.relu(h_new @ p['adv_W1'] + p['adv_b1']) @ p['adv_W2'] + p['adv_b2']).reshape(-1, NA, NO)
    q = val[:, None, :] + (adv - jnp.mean(adv, axis=1, keepdims=True))
    q = jnp.nan_to_num(q, nan=0.0, posinf=1e6, neginf=-1e6)
    q = jnp.clip(q, -100.0, 100.0)
    return q, h_new, c_new


def kernel(node_features, edge_index, h_prev, c_prev, w, params):
    p = params
    src = edge_index[0]
    dst = edge_index[1]
    zeros64 = jnp.zeros((APAD, HID), jnp.float32)
    zeros128 = jnp.zeros((APAD, _CW), jnp.float32)

    degp, _unused_parts = _deg_kernel(dst)
    d0 = degp[:N].reshape(N, 1)
    d1 = degp[HPAD:HPAD + N].reshape(N, 1)

    h1, xs1, dinv, dinv2 = _tc_a(node_features, d0, d1,
                                 p['feat_W'], p['feat_b'], p['g1_W'])
    S1 = _gcn_agg_kernel(xs1, src, dst, zeros64)
    h2, xs2 = _tc_b(S1[:N], S1[NPAD:NPAD + N], h1, dinv, dinv2,
                    p['g1_b'], p['g2_W'])
    S2 = _gcn_agg_kernel(xs2, src, dst, zeros64)
    q, k, v, skip = _tc_c(S2[:N], S2[NPAD:NPAD + N], h2, dinv, dinv2,
                          p['g2_b'], p)

    cat = _attn_kernel(q, k, v, src, dst, zeros128)

    ge = _tc_d(cat[:N], cat[NPAD:NPAD + N], skip) * (1.0 / N)
    return _head(ge, h_prev, c_prev, w, p)


# speedup vs baseline: 1.1051x; 1.1051x over previous
import functools

import jax
import jax.numpy as jnp
from jax import lax
from jax.experimental import pallas as pl
from jax.experimental.pallas import tpu as pltpu
from jax.experimental.pallas import tpu_sc as plsc

N = 100000
E = 1600000
RAW, EMB, HID, HEADS, LSTMH, NA, NO = 11, 32, 64, 8, 64, 7, 2
DH = HID // HEADS

ROWS = 4000
GRID = N // ROWS

NC, NS = 2, 16
NPART = 16
PSIZE = 6256
PCHUNK = 392
APAD = NS * PCHUNK
PT1 = 280
PT2 = 96
DUMP = PSIZE + 4
EB = 2000
EPW = E // NS
NBLK = EPW // EB
VPB = EB // 16
HPAD = 100096
HCH = HPAD // NS

_MESH = plsc.VectorSubcoreMesh(core_axis_name="c", subcore_axis_name="s",
                               num_cores=NC, num_subcores=NS)
_SC_PARAMS = pltpu.CompilerParams(needs_layout_passes=False,
                                  use_tc_tiling_on_sc=False)


@functools.partial(
    pl.kernel,
    out_type=[jax.ShapeDtypeStruct((NC * HPAD,), jnp.float32),
              jax.ShapeDtypeStruct((NC * NS * HPAD,), jnp.float32)],
    mesh=_MESH,
    compiler_params=_SC_PARAMS,
    scratch_types=dict(
        hist=pltpu.VMEM((HPAD,), jnp.float32),
        dstv=pltpu.VMEM((EB,), jnp.int32),
        accv=pltpu.VMEM((HCH,), jnp.float32),
        tmpv=pltpu.VMEM((HCH,), jnp.float32),
    ),
)
def _deg_kernel(dst_hbm, degp_hbm, part_hbm, hist, dstv, accv, tmpv):
    cid = lax.axis_index("c")
    sid = lax.axis_index("s")
    wid = sid * NC + cid
    zeros16 = jnp.zeros((16,), jnp.float32)
    ones16 = jnp.ones((16,), jnp.float32)

    def zero_body(i, _):
        hist[pl.ds(i * 16, 16)] = zeros16
        return 0

    lax.fori_loop(0, HPAD // 16, zero_body, 0)

    epw = E // (NC * NS)

    def blk_body(b, _):
        pltpu.sync_copy(dst_hbm.at[pl.ds(wid * epw + b * EB, EB)], dstv)

        def vec_body(t, _):
            d16 = dstv[pl.ds(t * 16, 16)]
            plsc.addupdate_scatter(hist, [d16], ones16)
            return 0

        return lax.fori_loop(0, VPB, vec_body, 0)

    lax.fori_loop(0, epw // EB, blk_body, 0)

    pltpu.sync_copy(hist, part_hbm.at[pl.ds((cid * NS + sid) * HPAD, HPAD)])
    plsc.subcore_barrier()

    col = sid * HCH
    pltpu.sync_copy(part_hbm.at[pl.ds(cid * NS * HPAD + col, HCH)], accv)
    for j in range(1, NS):
        pltpu.sync_copy(part_hbm.at[pl.ds((cid * NS + j) * HPAD + col, HCH)],
                        tmpv)

        def add_body(v, _):
            accv[pl.ds(v * 16, 16)] = accv[pl.ds(v * 16, 16)] + tmpv[pl.ds(v * 16, 16)]
            return 0

        lax.fori_loop(0, HCH // 16, add_body, 0)
    pltpu.sync_copy(accv, degp_hbm.at[pl.ds(cid * HPAD + col, HCH)])


@functools.partial(
    pl.kernel,
    out_type=jax.ShapeDtypeStruct((N, HID), jnp.float32),
    mesh=_MESH,
    compiler_params=_SC_PARAMS,
    scratch_types=dict(
        srcv=pltpu.VMEM((EB,), jnp.int32),
        dstv=pltpu.VMEM((EB,), jnp.int32),
        stage_s=pltpu.VMEM((160,), jnp.int32),
        stage_o=pltpu.VMEM((160,), jnp.int32),
        gidx=pltpu.VMEM((1, 128), jnp.int32),
        goff=pltpu.VMEM((1, 128), jnp.int32),
        rows=pltpu.VMEM((128, HID), jnp.float32),
        acc=pltpu.VMEM_SHARED((APAD, HID), jnp.float32),
    ),
)
def _gcn_agg_kernel(xs_hbm, src_hbm, dst_hbm, zeros_hbm, out_hbm,
                    srcv, dstv, stage_s, stage_o, gidx, goff, rows, acc):
    cid = lax.axis_index("c")
    sid = lax.axis_index("s")
    lanes = lax.iota(jnp.int32, 16)

    for cpart in range(NPART // NC):
        part = cpart * NC + cid
        base = part * PSIZE

        pltpu.sync_copy(zeros_hbm.at[pl.ds(sid * PCHUNK, PCHUNK)],
                        acc.at[pl.ds(sid * PCHUNK, PCHUNK)])
        plsc.subcore_barrier()

        def fire(n_valid):
            for i in range(8):
                sv = stage_s[pl.ds(i * 16, 16)]
                ov = stage_o[pl.ds(i * 16, 16)]
                valid = (lanes + (i * 16)) < n_valid
                gidx[0, pl.ds(i * 16, 16)] = jnp.where(valid, sv, 0)
                goff[0, pl.ds(i * 16, 16)] = jnp.where(valid, ov, DUMP)
            pltpu.sync_copy(xs_hbm.at[gidx.at[0]], rows)
            pltpu.sync_copy(rows, acc.at[goff.at[0]], add=True)

        def blk_body(b, ptr):
            ebase = sid * EPW + b * EB
            pltpu.sync_copy(src_hbm.at[pl.ds(ebase, EB)], srcv)
            pltpu.sync_copy(dst_hbm.at[pl.ds(ebase, EB)], dstv)

            def vec_body(t, ptr):
                s16 = srcv[pl.ds(t * 16, 16)]
                d16 = dstv[pl.ds(t * 16, 16)]
                m = (d16 >= base) & (d16 < base + PSIZE)
                plsc.store_compressed(stage_s.at[pl.ds(ptr, 16)], s16, mask=m)
                plsc.store_compressed(stage_o.at[pl.ds(ptr, 16)], d16 - base,
                                      mask=m)
                ptr = ptr + jnp.sum(m.astype(jnp.int32))

                def do_fire(p):
                    fire(jnp.int32(128))
                    rs = stage_s[pl.ds(128, 16)]
                    ro = stage_o[pl.ds(128, 16)]
                    stage_s[pl.ds(0, 16)] = rs
                    stage_o[pl.ds(0, 16)] = ro
                    return p - 128

                return lax.cond(ptr >= 128, do_fire, lambda p: p, ptr)

            return lax.fori_loop(0, VPB, vec_body, ptr)

        ptr = lax.fori_loop(0, NBLK, blk_body, jnp.int32(0))

        def tail(p):
            fire(p)
            return jnp.int32(0)

        ptr = lax.cond(ptr > 0, tail, lambda p: jnp.int32(0), ptr)
        plsc.subcore_barrier()

        @pl.when(sid < NS - 1)
        def _():
            pltpu.sync_copy(acc.at[pl.ds(sid * PCHUNK, PCHUNK)],
                            out_hbm.at[pl.ds(base + sid * PCHUNK, PCHUNK)])

        @pl.when(sid == NS - 1)
        def _():
            pltpu.sync_copy(acc.at[pl.ds((NS - 1) * PCHUNK, PT1)],
                            out_hbm.at[pl.ds(base + (NS - 1) * PCHUNK, PT1)])

        @pl.when((sid == NS - 1) & (base + PSIZE <= N))
        def _():
            off = (NS - 1) * PCHUNK + PT1
            pltpu.sync_copy(acc.at[pl.ds(off, PT2)],
                            out_hbm.at[pl.ds(base + off, PT2)])

        plsc.subcore_barrier()


_CW = 80

_GDN = lax.GatherDimensionNumbers(offset_dims=(), collapsed_slice_dims=(0,),
                                  start_index_map=(0,))


def _perm(x, idx):
    return lax.gather(x, idx.reshape(16, 1), _GDN, slice_sizes=(1,),
                      mode=lax.GatherScatterMode.PROMISE_IN_BOUNDS)


@functools.partial(
    pl.kernel,
    out_type=jax.ShapeDtypeStruct((N, _CW), jnp.float32),
    mesh=_MESH,
    compiler_params=_SC_PARAMS,
    scratch_types=dict(
        srcv=pltpu.VMEM((EB,), jnp.int32),
        dstv=pltpu.VMEM((EB,), jnp.int32),
        stage_s=pltpu.VMEM((160,), jnp.int32),
        stage_d=pltpu.VMEM((160,), jnp.int32),
        gsrc=pltpu.VMEM((1, 128), jnp.int32),
        gdst=pltpu.VMEM((1, 128), jnp.int32),
        goff=pltpu.VMEM((1, 128), jnp.int32),
        qrows=pltpu.VMEM((128, HID), jnp.float32),
        krows=pltpu.VMEM((128, HID), jnp.float32),
        vrows=pltpu.VMEM((128, HID), jnp.float32),
        cat=pltpu.VMEM((128, _CW), jnp.float32),
        acc=pltpu.VMEM_SHARED((APAD, _CW), jnp.float32),
    ),
)
def _attn_kernel(q_hbm, k_hbm, v_hbm, src_hbm, dst_hbm, zeros_hbm, out_hbm,
                 srcv, dstv, stage_s, stage_d, gsrc, gdst, goff,
                 qrows, krows, vrows, cat, acc):
    cid = lax.axis_index("c")
    sid = lax.axis_index("s")
    lanes = lax.iota(jnp.int32, 16)
    scale = jnp.float32(1.0 / (DH ** 0.5))
    zeros16 = jnp.zeros((16,), jnp.float32)
    perm_sw4 = lanes ^ 4
    perm_sw2 = lanes ^ 2
    perm_sw1 = lanes ^ 1
    perm_den = (lanes & 1) * 8

    for cpart in range(NPART // NC):
        part = cpart * NC + cid
        base = part * PSIZE

        pltpu.sync_copy(zeros_hbm.at[pl.ds(sid * PCHUNK, PCHUNK)],
                        acc.at[pl.ds(sid * PCHUNK, PCHUNK)])
        plsc.subcore_barrier()

        def fire(n_valid):
            for i in range(8):
                sv = stage_s[pl.ds(i * 16, 16)]
                dv = stage_d[pl.ds(i * 16, 16)]
                valid = (lanes + (i * 16)) < n_valid
                gsrc[0, pl.ds(i * 16, 16)] = jnp.where(valid, sv, 0)
                gdst[0, pl.ds(i * 16, 16)] = jnp.where(valid, dv, 0)
                goff[0, pl.ds(i * 16, 16)] = jnp.where(valid, dv - base, DUMP)
            pltpu.sync_copy(q_hbm.at[gdst.at[0]], qrows)
            pltpu.sync_copy(k_hbm.at[gsrc.at[0]], krows)
            pltpu.sync_copy(v_hbm.at[gsrc.at[0]], vrows)

            def edge_body(e, _):
                exs = []
                for j in range(4):
                    qj = qrows[e, pl.ds(j * 16, 16)]
                    kj = krows[e, pl.ds(j * 16, 16)]
                    t = qj * kj
                    t = t + _perm(t, perm_sw4)
                    t = t + _perm(t, perm_sw2)
                    t = t + _perm(t, perm_sw1)
                    ej = jnp.exp(jnp.minimum(t * scale, 70.0))
                    exs.append(ej)
                    cat[e, pl.ds(j * 16, 16)] = ej * vrows[e, pl.ds(j * 16, 16)]
                den = zeros16
                for j in range(4):
                    tj = _perm(exs[j], perm_den)
                    mj = (lanes >= 2 * j) & (lanes < 2 * j + 2)
                    den = jnp.where(mj, tj, den)
                cat[e, pl.ds(64, 16)] = den
                return 0

            lax.fori_loop(0, 128, edge_body, 0)
            pltpu.sync_copy(cat, acc.at[goff.at[0]], add=True)

        def blk_body(b, ptr):
            ebase = sid * EPW + b * EB
            pltpu.sync_copy(src_hbm.at[pl.ds(ebase, EB)], srcv)
            pltpu.sync_copy(dst_hbm.at[pl.ds(ebase, EB)], dstv)

            def vec_body(t, ptr):
                s16 = srcv[pl.ds(t * 16, 16)]
                d16 = dstv[pl.ds(t * 16, 16)]
                m = (d16 >= base) & (d16 < base + PSIZE)
                plsc.store_compressed(stage_s.at[pl.ds(ptr, 16)], s16, mask=m)
                plsc.store_compressed(stage_d.at[pl.ds(ptr, 16)], d16, mask=m)
                ptr = ptr + jnp.sum(m.astype(jnp.int32))

                def do_fire(p):
                    fire(jnp.int32(128))
                    rs = stage_s[pl.ds(128, 16)]
                    rd = stage_d[pl.ds(128, 16)]
                    stage_s[pl.ds(0, 16)] = rs
                    stage_d[pl.ds(0, 16)] = rd
                    return p - 128

                return lax.cond(ptr >= 128, do_fire, lambda p: p, ptr)

            return lax.fori_loop(0, VPB, vec_body, ptr)

        ptr = lax.fori_loop(0, NBLK, blk_body, jnp.int32(0))

        def tail(p):
            fire(p)
            return jnp.int32(0)

        ptr = lax.cond(ptr > 0, tail, lambda p: jnp.int32(0), ptr)
        plsc.subcore_barrier()

        @pl.when(sid < NS - 1)
        def _():
            pltpu.sync_copy(acc.at[pl.ds(sid * PCHUNK, PCHUNK)],
                            out_hbm.at[pl.ds(base + sid * PCHUNK, PCHUNK)])

        @pl.when(sid == NS - 1)
        def _():
            pltpu.sync_copy(acc.at[pl.ds((NS - 1) * PCHUNK, PT1)],
                            out_hbm.at[pl.ds(base + (NS - 1) * PCHUNK, PT1)])

        @pl.when((sid == NS - 1) & (base + PSIZE <= N))
        def _():
            off = (NS - 1) * PCHUNK + PT1
            pltpu.sync_copy(acc.at[pl.ds(off, PT2)],
                            out_hbm.at[pl.ds(base + off, PT2)])

        plsc.subcore_barrier()


def _tc_a_body(nf_ref, d0_ref, d1_ref, w1_ref, b1_ref, w2_ref,
               h1_ref, xs_ref, dinv_ref, dinv2_ref):
    x = jnp.maximum(nf_ref[...] @ w1_ref[...] + b1_ref[...], 0.0)
    h1 = x @ w2_ref[...]
    deg = 1.0 + d0_ref[...] + d1_ref[...]
    dinv = lax.rsqrt(deg)
    h1_ref[...] = h1
    xs_ref[...] = dinv * h1
    dinv_ref[...] = dinv
    dinv2_ref[...] = dinv * dinv


def _tc_a(nf, d0, d1, w1, b1, w2):
    col = pl.BlockSpec((ROWS, 1), lambda i: (i, 0))
    row = pl.BlockSpec((ROWS, HID), lambda i: (i, 0))
    return pl.pallas_call(
        _tc_a_body,
        grid=(GRID,),
        in_specs=[
            pl.BlockSpec((ROWS, RAW), lambda i: (i, 0)), col, col,
            pl.BlockSpec((RAW, EMB), lambda i: (0, 0)),
            pl.BlockSpec((1, EMB), lambda i: (0, 0)),
            pl.BlockSpec((EMB, HID), lambda i: (0, 0)),
        ],
        out_specs=[row, row, col, col],
        out_shape=[
            jax.ShapeDtypeStruct((N, HID), jnp.float32),
            jax.ShapeDtypeStruct((N, HID), jnp.float32),
            jax.ShapeDtypeStruct((N, 1), jnp.float32),
            jax.ShapeDtypeStruct((N, 1), jnp.float32),
        ],
    )(nf, d0, d1, w1, b1.reshape(1, EMB), w2)


def _tc_b_body(s_ref, h_ref, dinv_ref, d2_ref, b_ref, w_ref, h2_ref, xs_ref):
    t = jnp.maximum(dinv_ref[...] * s_ref[...] + d2_ref[...] * h_ref[...]
                    + b_ref[...], 0.0)
    h2 = t @ w_ref[...]
    h2_ref[...] = h2
    xs_ref[...] = dinv_ref[...] * h2


def _tc_b(S, h, dinv, dinv2, b, w2):
    col = pl.BlockSpec((ROWS, 1), lambda i: (i, 0))
    row = pl.BlockSpec((ROWS, HID), lambda i: (i, 0))
    return pl.pallas_call(
        _tc_b_body,
        grid=(GRID,),
        in_specs=[row, row, col, col,
                  pl.BlockSpec((1, HID), lambda i: (0, 0)),
                  pl.BlockSpec((HID, HID), lambda i: (0, 0))],
        out_specs=[row, row],
        out_shape=[jax.ShapeDtypeStruct((N, HID), jnp.float32)] * 2,
    )(S, h, dinv, dinv2, b.reshape(1, HID), w2)


def _tc_c_body(s_ref, h_ref, dinv_ref, d2_ref, gb_ref, wq_ref, wk_ref, wv_ref,
               ws_ref, bq_ref, bk_ref, bv_ref, bs_ref,
               q_ref, k_ref, v_ref, sk_ref):
    x = jnp.maximum(dinv_ref[...] * s_ref[...] + d2_ref[...] * h_ref[...]
                    + gb_ref[...], 0.0)
    q_ref[...] = x @ wq_ref[...] + bq_ref[...]
    k_ref[...] = x @ wk_ref[...] + bk_ref[...]
    v_ref[...] = x @ wv_ref[...] + bv_ref[...]
    sk_ref[...] = x @ ws_ref[...] + bs_ref[...]


def _tc_c(S, h, dinv, dinv2, gb, p):
    col = pl.BlockSpec((ROWS, 1), lambda i: (i, 0))
    row = pl.BlockSpec((ROWS, HID), lambda i: (i, 0))
    wspec = pl.BlockSpec((HID, HID), lambda i: (0, 0))
    bspec = pl.BlockSpec((1, HID), lambda i: (0, 0))
    return pl.pallas_call(
        _tc_c_body,
        grid=(GRID,),
        in_specs=[row, row, col, col, bspec,
                  wspec, wspec, wspec, wspec, bspec, bspec, bspec, bspec],
        out_specs=[row, row, row, row],
        out_shape=[jax.ShapeDtypeStruct((N, HID), jnp.float32)] * 4,
    )(S, h, dinv, dinv2, gb.reshape(1, HID),
      p['Wq'], p['Wk'], p['Wv'], p['Wskip'],
      p['bq'].reshape(1, HID), p['bk'].reshape(1, HID),
      p['bv'].reshape(1, HID), p['bskip'].reshape(1, HID))


def _tc_d_body(cat_ref, sk_ref, rep_ref, out_ref):
    @pl.when(pl.program_id(0) == 0)
    def _():
        out_ref[...] = jnp.zeros_like(out_ref)
    num = cat_ref[:, :HID]
    den = cat_ref[:, HID:HID + HEADS] @ rep_ref[...]
    attn = num / (den + 1e-16)
    out_ref[...] += jnp.sum(attn + sk_ref[...], axis=0, keepdims=True)


def _tc_d(cat, skip):
    rep = jnp.repeat(jnp.eye(HEADS, dtype=jnp.float32), DH, axis=1)
    return pl.pallas_call(
        _tc_d_body,
        grid=(GRID,),
        in_specs=[
            pl.BlockSpec((ROWS, _CW), lambda i: (i, 0)),
            pl.BlockSpec((ROWS, HID), lambda i: (i, 0)),
            pl.BlockSpec((HEADS, HID), lambda i: (0, 0)),
        ],
        out_specs=pl.BlockSpec((1, HID), lambda i: (0, 0)),
        out_shape=jax.ShapeDtypeStruct((1, HID), jnp.float32),
    )(cat, skip, rep)


def _head(ge, h_prev, c_prev, w, p):
    ww = w.reshape(1, -1)
    g = ww @ p['film_gW'] + p['film_gb']
    b = ww @ p['film_bW'] + p['film_bb']
    ge = ge * (1.0 + g) + b
    gates = ge @ p['lstm_Wih'] + p['lstm_bih'] + h_prev @ p['lstm_Whh'] + p['lstm_bhh']
    i, f, gg, o = jnp.split(gates, 4, axis=-1)
    i = jax.nn.sigmoid(i)
    f = jax.nn.sigmoid(f)
    gg = jnp.tanh(gg)
    o = jax.nn.sigmoid(o)
    c_new = f * c_prev + i * gg
    h_new = o * jnp.tanh(c_new)
    mu = jnp.mean(h_new, axis=-1, keepdims=True)
    var = jnp.var(h_new, axis=-1, keepdims=True)
    h_new = (h_new - mu) / jnp.sqrt(var + 1e-5) * p['ln_g'] + p['ln_b']
    c_new = jnp.clip(c_new, -1e6, 1e6)
    val = jax.nn.relu(h_new @ p['val_W1'] + p['val_b1']) @ p['val_W2'] + p['val_b2']
    adv = (jax.nn.relu(h_new @ p['adv_W1'] + p['adv_b1']) @ p['adv_W2'] + p['adv_b2']).reshape(-1, NA, NO)
    q = val[:, None, :] + (adv - jnp.mean(adv, axis=1, keepdims=True))
    q = jnp.nan_to_num(q, nan=0.0, posinf=1e6, neginf=-1e6)
    q = jnp.clip(q, -100.0, 100.0)
    return q, h_new, c_new


def kernel(node_features, edge_index, h_prev, c_prev, w, params):
    p = params
    src = edge_index[0]
    dst = edge_index[1]
    zeros64 = jnp.zeros((APAD, HID), jnp.float32)

    degp, _unused_parts = _deg_kernel(dst)
    d0 = degp[:N].reshape(N, 1)
    d1 = degp[HPAD:HPAD + N].reshape(N, 1)

    h1, xs1, dinv, dinv2 = _tc_a(node_features, d0, d1,
                                 p['feat_W'], p['feat_b'], p['g1_W'])
    S1 = _gcn_agg_kernel(xs1, src, dst, zeros64)
    h2, xs2 = _tc_b(S1, h1, dinv, dinv2, p['g1_b'], p['g2_W'])
    S2 = _gcn_agg_kernel(xs2, src, dst, zeros64)
    q, k, v, skip = _tc_c(S2, h2, dinv, dinv2, p['g2_b'], p)

    zeros80 = jnp.zeros((APAD, _CW), jnp.float32)
    cat = _attn_kernel(q, k, v, src, dst, zeros80)

    ge = _tc_d(cat, skip) * (1.0 / N)
    return _head(ge, h_prev, c_prev, w, p)
